# Initial kernel scaffold; baseline (speedup 1.0000x reference)
import jax
import jax.numpy as jnp
from jax.experimental import pallas as pl


def kernel(x, token_table, pos_table):
    raise NotImplementedError("write your pallas kernel here")


# speedup vs baseline: 2.6601x; 2.6601x over previous
import functools

import jax
import jax.numpy as jnp
from jax import lax
from jax.experimental import pallas as pl
from jax.experimental.pallas import tpu as pltpu
from jax.experimental.pallas import tpu_sc as plsc

D = 32
L = 16
GSZ = 128
C = 1280
G = C // GSZ


def _build_sc_kernel(n_rows: int, n_chunks: int):
    info = plsc.get_sparse_core_info()
    nc, ns = info.num_cores, info.num_subcores
    nw = nc * ns
    rows_per_worker = n_rows // nw
    mesh = plsc.VectorSubcoreMesh(core_axis_name="c", subcore_axis_name="s")

    @functools.partial(
        pl.kernel,
        out_type=jax.ShapeDtypeStruct((n_rows, D), jnp.float32),
        mesh=mesh,
        compiler_params=pltpu.CompilerParams(use_tc_tiling_on_sc=False),
        scratch_types=[
            pltpu.VMEM((G, GSZ), jnp.int32),
            pltpu.VMEM((G, GSZ), jnp.int32),
            pltpu.VMEM((G, GSZ), jnp.int32),
            pltpu.VMEM((C, D), jnp.float32),
            pltpu.VMEM((C, D), jnp.float32),
            pltpu.SemaphoreType.DMA,
        ],
    )
    def k(idx_hbm, colp_hbm, tok_hbm, pos_hbm, out_hbm,
          idx_v, pidx_v, colp_v, tokr_v, posr_v, sem):
        wid = lax.axis_index("s") * nc + lax.axis_index("c")

        def chunk_body(ci, carry):
            pltpu.sync_copy(idx_hbm.at[wid, ci], idx_v)
            pltpu.sync_copy(colp_hbm.at[ci], colp_v)

            def pid_body(g, c2):
                for j in range(GSZ // L):
                    sl = pl.ds(j * L, L)
                    iv = idx_v[g, sl]
                    cp = colp_v[g, sl]
                    pidx_v[g, sl] = jnp.where(
                        iv > 0, cp, jnp.zeros((L,), jnp.int32))
                return c2
            lax.fori_loop(0, G, pid_body, 0)

            copies = []
            for g in range(G):
                copies.append(pltpu.async_copy(
                    tok_hbm.at[idx_v.at[g]],
                    tokr_v.at[pl.ds(g * GSZ, GSZ)], sem))
                copies.append(pltpu.async_copy(
                    pos_hbm.at[pidx_v.at[g]],
                    posr_v.at[pl.ds(g * GSZ, GSZ)], sem))
            for cp in copies:
                cp.wait()

            def add_body(r, c2):
                for u in range(8):
                    row = r * 8 + u
                    for h in range(D // L):
                        sl = pl.ds(h * L, L)
                        tokr_v[row, sl] = tokr_v[row, sl] + posr_v[row, sl]
                return c2
            lax.fori_loop(0, C // 8, add_body, 0)

            pltpu.sync_copy(
                tokr_v, out_hbm.at[pl.ds(wid * rows_per_worker + ci * C, C)])
            return carry
        lax.fori_loop(0, n_chunks, chunk_body, 0)

    return k


@jax.jit
def kernel(x, token_table, pos_table):
    b, xlen = x.shape
    n_rows = b * xlen
    info = plsc.get_sparse_core_info()
    nw = info.num_cores * info.num_subcores
    rows_per_worker = n_rows // nw
    n_chunks = rows_per_worker // C

    idx = x.reshape(-1).astype(jnp.int32).reshape(nw, n_chunks, G, GSZ)
    colp = ((jnp.arange(rows_per_worker, dtype=jnp.int32) % xlen) + 1
            ).reshape(n_chunks, G, GSZ)

    out = _build_sc_kernel(n_rows, n_chunks)(
        idx, colp, token_table, pos_table)
    return out.reshape(b, xlen, D)
